# baseline (speedup 1.0000x reference)
import jax
import jax.numpy as jnp
from jax import lax
from jax.experimental import pallas as pl
from jax.experimental.pallas import tpu as pltpu
from jax.experimental.pallas import tpu_sc as plsc

B, T, F = 4096, 200, 64
NC, NS, L = 2, 16, 16
NW = NC * NS
CPW = B // NW


def _sc_body(in_hbm, out_hbm, slab_v, sem0, sem1, osem):
    wid = lax.axis_index("s") * NC + lax.axis_index("c")
    base = wid * CPW
    H = F // 2

    cp0 = pltpu.async_copy(
        in_hbm.at[T - 1, pl.ds(0, H), pl.ds(base, CPW)], slab_v.at[pl.ds(0, H)], sem0
    )
    cp1 = pltpu.async_copy(
        in_hbm.at[T - 1, pl.ds(H, H), pl.ds(base, CPW)], slab_v.at[pl.ds(H, H)], sem1
    )

    table = lax.iota(jnp.int32, L)
    dnums = lax.GatherDimensionNumbers(
        offset_dims=(), collapsed_slice_dims=(0,), start_index_map=(0,)
    )

    def body(r, carry):
        for j in range(CPW // L):
            idx = slab_v[r, pl.ds(j * L, L)]
            slab_v[r, pl.ds(j * L, L)] = lax.gather(
                table, idx[:, None], dnums, slice_sizes=(1,),
                mode=lax.GatherScatterMode.PROMISE_IN_BOUNDS,
            )
        return carry

    cp0.wait()
    lax.fori_loop(0, H, body, 0)
    ocp0 = pltpu.async_copy(
        slab_v.at[pl.ds(0, H)], out_hbm.at[pl.ds(0, H), pl.ds(base, CPW)], osem
    )
    cp1.wait()
    lax.fori_loop(H, F, body, 0)
    ocp1 = pltpu.async_copy(
        slab_v.at[pl.ds(H, H)], out_hbm.at[pl.ds(H, H), pl.ds(base, CPW)], osem
    )
    ocp0.wait()
    ocp1.wait()


@jax.jit
def kernel(indices):
    tview = jnp.transpose(indices, (1, 2, 0))
    run = pl.kernel(
        _sc_body,
        out_type=jax.ShapeDtypeStruct((F, B), jnp.int32),
        mesh=plsc.VectorSubcoreMesh(core_axis_name="c", subcore_axis_name="s"),
        scratch_types=[
            pltpu.VMEM((F, CPW), jnp.int32),
            pltpu.SemaphoreType.DMA,
            pltpu.SemaphoreType.DMA,
            pltpu.SemaphoreType.DMA,
        ],
    )
    return jnp.transpose(run(tview))
